# baseline (speedup 1.0000x reference)
import functools

import jax
import jax.numpy as jnp
from jax import lax
from jax.experimental import pallas as pl
from jax.experimental.pallas import tpu as pltpu
from jax.experimental.pallas import tpu_sc as plsc

E = 16
K = 8
D = 2048
I = 1024
N_GROUP = 4
GSZ = E // N_GROUP
TOPK_GROUP = 2
ROUTED_SCALING = 2.5
CAP = 640
ECAP = E * CAP
TRASH = ECAP
TIDS_N = ECAP + 8
T = 8192

NC, NS, L = 2, 16, 16
NW = NC * NS

INTERP = False

_NEG = -1e30


_BT = 512


def _router_body(x_ref, rw_ref, eb_ref, slot_ref, w_ref, cnt_ref):
    bt = _BT
    x = x_ref[...]
    logits = lax.dot_general(x, rw_ref[...], (((1,), (1,)), ((), ())),
                             preferred_element_type=jnp.float32)
    scores = jax.nn.sigmoid(logits)
    sfc = scores + eb_ref[...]

    li = lax.broadcasted_iota(jnp.int32, (bt, E), 1)
    lg = li // GSZ

    gs = []
    for g in range(N_GROUP):
        mg = lg == g
        vals = jnp.where(mg, sfc, _NEG)
        m1 = jnp.max(vals, axis=1, keepdims=True)
        pos1 = jnp.min(jnp.where(vals == m1, li, 99), axis=1, keepdims=True)
        m2 = jnp.max(jnp.where(li == pos1, _NEG, vals), axis=1, keepdims=True)
        gs.append(m1 + m2)

    best1 = gs[0]
    gi1 = jnp.zeros_like(gs[0], dtype=jnp.int32)
    for g in range(1, N_GROUP):
        b = gs[g] > best1
        best1 = jnp.where(b, gs[g], best1)
        gi1 = jnp.where(b, g, gi1)
    best2 = jnp.full_like(best1, _NEG)
    gi2 = jnp.full_like(gi1, -1)
    for g in range(N_GROUP):
        b = (gi1 != g) & (gs[g] > best2)
        best2 = jnp.where(b, gs[g], best2)
        gi2 = jnp.where(b, g, gi2)
    gmask = (lg == gi1) | (lg == gi2)

    masked = jnp.where(gmask, sfc, 0.0)

    cur = masked
    sel = jnp.zeros((bt, E), dtype=jnp.bool_)
    pos_list = []
    for _ in range(K):
        m = jnp.max(cur, axis=1, keepdims=True)
        pos = jnp.min(jnp.where(cur == m, li, 99), axis=1, keepdims=True)
        pos_list.append(pos)
        hit = li == pos
        sel = sel | hit
        cur = jnp.where(hit, _NEG, cur)

    selr = sel.astype(jnp.float32)
    wsum = jnp.sum(jnp.where(sel, scores, 0.0), axis=1, keepdims=True)
    inv = ROUTED_SCALING / (wsum + 1e-20)

    @pl.when(pl.program_id(0) == 0)
    def _():
        cnt_ref[...] = jnp.zeros_like(cnt_ref)

    r0 = lax.broadcasted_iota(jnp.int32, (bt, bt), 0)
    r1 = lax.broadcasted_iota(jnp.int32, (bt, bt), 1)
    tri = (r0 > r1).astype(jnp.float32)
    prefix = jnp.dot(tri, selr, preferred_element_type=jnp.float32)
    rank_f = prefix + cnt_ref[...]
    cnt_ref[...] = cnt_ref[...] + jnp.sum(selr, axis=0, keepdims=True)

    slot_cols = []
    w_cols = []
    for pos in pos_list:
        hit = li == pos
        rank_j = jnp.sum(jnp.where(hit, rank_f, 0.0), axis=1, keepdims=True)
        score_j = jnp.sum(jnp.where(hit, scores, 0.0), axis=1, keepdims=True)
        keep_j = rank_j < CAP
        w_cols.append(jnp.where(keep_j, score_j * inv, 0.0))
        slot_cols.append(jnp.where(keep_j, pos * CAP + rank_j.astype(jnp.int32),
                                   TRASH))
    slot_ref[...] = jnp.concatenate(slot_cols, axis=1)
    w_ref[...] = jnp.concatenate(w_cols, axis=1)


def _router(x, router_weight, e_bias):
    nblk = T // _BT
    return pl.pallas_call(
        _router_body,
        grid=(nblk,),
        in_specs=[
            pl.BlockSpec((_BT, D), lambda i: (i, 0)),
            pl.BlockSpec((E, D), lambda i: (0, 0)),
            pl.BlockSpec((1, E), lambda i: (0, 0)),
        ],
        out_specs=[
            pl.BlockSpec((_BT, K), lambda i: (i, 0)),
            pl.BlockSpec((_BT, K), lambda i: (i, 0)),
        ],
        out_shape=[
            jax.ShapeDtypeStruct((T, K), jnp.int32),
            jax.ShapeDtypeStruct((T, K), jnp.float32),
        ],
        scratch_shapes=[pltpu.VMEM((1, E), jnp.float32)],
        interpret=INTERP,
    )(x, router_weight, e_bias.reshape(1, E))


_SCA_CH = (T * K) // NW // 128

_sc_mesh = functools.partial(
    plsc.VectorSubcoreMesh, core_axis_name="c", subcore_axis_name="s")


def _sc_scatter_tids(slot_r, tval_r):
    @functools.partial(
        pl.kernel,
        mesh=_sc_mesh(),
        out_type=jax.ShapeDtypeStruct((TIDS_N,), jnp.int32),
        scratch_types=[
            pltpu.VMEM((_SCA_CH, 128), jnp.int32),
            pltpu.VMEM((_SCA_CH, 128), jnp.int32),
            pltpu.SemaphoreType.DMA,
        ],
    )
    def sca(slot_hbm, tval_hbm, tids_hbm, idx_v, val_v, sem):
        wid = lax.axis_index("s") * NC + lax.axis_index("c")
        pltpu.sync_copy(slot_hbm.at[wid], idx_v)
        pltpu.sync_copy(tval_hbm.at[wid], val_v)
        for j in range(_SCA_CH):
            pltpu.async_copy(val_v.at[j], tids_hbm.at[idx_v.at[j]], sem).wait()

    return sca(slot_r, tval_r)


_SCB_RPW = ECAP // NW
_SCB_CH = 16


def _sc_gather_x(tids, x):
    @functools.partial(
        pl.kernel,
        mesh=_sc_mesh(),
        out_type=jax.ShapeDtypeStruct((ECAP, D), jnp.float32),
        scratch_types=[
            pltpu.VMEM((_SCB_CH,), jnp.int32),
            pltpu.VMEM((_SCB_CH, D), jnp.float32),
            pltpu.SemaphoreType.DMA,
        ],
    )
    def scb(tids_hbm, x_hbm, xb_hbm, idx_v, rows_v, sem):
        wid = lax.axis_index("s") * NC + lax.axis_index("c")
        base = wid * _SCB_RPW

        def body(i, _):
            b = pl.multiple_of(base + i * _SCB_CH, _SCB_CH)
            pltpu.sync_copy(tids_hbm.at[pl.ds(b, _SCB_CH)], idx_v)
            v = idx_v[...]
            idx_v[...] = jnp.minimum(jnp.maximum(v, 0), T - 1)
            pltpu.async_copy(x_hbm.at[idx_v], rows_v, sem).wait()
            pltpu.sync_copy(rows_v, xb_hbm.at[pl.ds(b, _SCB_CH)])
            return 0

        lax.fori_loop(0, _SCB_RPW // _SCB_CH, body, 0)

    return scb(tids, x)


_NI = 2
_BI = I // _NI


def _mlp_body(xb_ref, wg_ref, wu_ref, wd_ref, y_ref):
    xb = xb_ref[0]
    g = jnp.dot(xb, wg_ref[0], preferred_element_type=jnp.float32)
    u = jnp.dot(xb, wu_ref[0], preferred_element_type=jnp.float32)
    h = g * jax.nn.sigmoid(g) * u
    part = jnp.dot(h, wd_ref[0], preferred_element_type=jnp.float32)

    @pl.when(pl.program_id(1) == 0)
    def _():
        y_ref[0] = part

    @pl.when(pl.program_id(1) != 0)
    def _():
        y_ref[0] = y_ref[0] + part


def _mlp(xb, W_gate, W_up, W_down):
    return pl.pallas_call(
        _mlp_body,
        grid=(E, _NI),
        in_specs=[
            pl.BlockSpec((1, CAP, D), lambda e, s: (e, 0, 0)),
            pl.BlockSpec((1, D, _BI), lambda e, s: (e, 0, s)),
            pl.BlockSpec((1, D, _BI), lambda e, s: (e, 0, s)),
            pl.BlockSpec((1, _BI, D), lambda e, s: (e, s, 0)),
        ],
        out_specs=pl.BlockSpec((1, CAP, D), lambda e, s: (e, 0, 0)),
        out_shape=jax.ShapeDtypeStruct((E, CAP, D), jnp.float32),
        interpret=INTERP,
    )(xb, W_gate, W_up, W_down)


_BT4 = 256


def _shared_body(x_ref, sg_ref, su_ref, sd_ref, o_ref):
    x = x_ref[...]
    g = lax.dot_general(x, sg_ref[...], (((1,), (1,)), ((), ())),
                        preferred_element_type=jnp.float32)
    u = lax.dot_general(x, su_ref[...], (((1,), (1,)), ((), ())),
                        preferred_element_type=jnp.float32)
    h = g * jax.nn.sigmoid(g) * u
    o_ref[...] = lax.dot_general(h, sd_ref[...], (((1,), (1,)), ((), ())),
                                 preferred_element_type=jnp.float32)


def _shared(x, shared_gate, shared_up, shared_down):
    SH = shared_gate.shape[0]
    return pl.pallas_call(
        _shared_body,
        grid=(T // _BT4,),
        in_specs=[
            pl.BlockSpec((_BT4, D), lambda i: (i, 0)),
            pl.BlockSpec((SH, D), lambda i: (0, 0)),
            pl.BlockSpec((SH, D), lambda i: (0, 0)),
            pl.BlockSpec((D, SH), lambda i: (0, 0)),
        ],
        out_specs=pl.BlockSpec((_BT4, D), lambda i: (i, 0)),
        out_shape=jax.ShapeDtypeStruct((T, D), jnp.float32),
        interpret=INTERP,
    )(x, shared_gate, shared_up, shared_down)


_SCC_TPW = T // NW
_SCC_IT = _SCC_TPW // 2


def _sc_combine(y, slot_r, w_r, shared_out):
    @functools.partial(
        pl.kernel,
        mesh=_sc_mesh(),
        out_type=jax.ShapeDtypeStruct((T, D), jnp.float32),
        scratch_types=[
            pltpu.VMEM((16,), jnp.int32),
            pltpu.VMEM((16,), jnp.float32),
            pltpu.VMEM((16, D), jnp.float32),
            pltpu.VMEM((2, D), jnp.float32),
            pltpu.SemaphoreType.DMA,
        ],
    )
    def scc(y_hbm, slot_hbm, w_hbm, sh_hbm, out_hbm, idx_v, w_v, rows_v,
            acc_v, sem):
        wid = lax.axis_index("s") * NC + lax.axis_index("c")
        tbase = wid * _SCC_TPW

        def body(i, _):
            pltpu.sync_copy(slot_hbm.at[wid, i], idx_v)
            pltpu.sync_copy(w_hbm.at[wid, i], w_v)
            v = idx_v[...]
            idx_v[...] = jnp.minimum(jnp.maximum(v, 0), ECAP - 1)
            pltpu.async_copy(y_hbm.at[idx_v], rows_v, sem).wait()
            t0 = pl.multiple_of(tbase + 2 * i, 2)
            pltpu.sync_copy(sh_hbm.at[pl.ds(t0, 2)], acc_v)
            w16 = w_v[...]
            _dn = lax.GatherDimensionNumbers(offset_dims=(),
                                             collapsed_slice_dims=(0,),
                                             start_index_map=(0,))
            for a in range(2):
                wsp = [lax.gather(w16,
                                  jnp.full((L, 1), a * K + j, jnp.int32),
                                  _dn, (1,),
                                  mode=lax.GatherScatterMode.PROMISE_IN_BOUNDS)
                       for j in range(K)]

                def cbody(c, _, a=a, wsp=wsp):
                    sl = pl.ds(pl.multiple_of(c * L, L), L)
                    acc = acc_v[a, sl]
                    for j in range(K):
                        acc = acc + wsp[j] * rows_v[a * K + j, sl]
                    acc_v[a, sl] = acc
                    return 0

                lax.fori_loop(0, D // L, cbody, 0)
            pltpu.sync_copy(acc_v, out_hbm.at[pl.ds(t0, 2)])
            return 0

        lax.fori_loop(0, _SCC_IT, body, 0)

    return scc(y, slot_r, w_r, shared_out)


def kernel(hidden_states, router_weight, e_bias, W_gate, W_up, W_down,
           shared_gate, shared_up, shared_down):
    B, S, Dm = hidden_states.shape
    x = hidden_states.reshape(T, Dm)

    slot8, w8 = _router(x, router_weight, e_bias)

    tvals = jnp.broadcast_to(jnp.arange(T, dtype=jnp.int32)[:, None], (T, K))
    tids = _sc_scatter_tids(slot8.reshape(NW, _SCA_CH, 128),
                            tvals.reshape(NW, _SCA_CH, 128))

    xb = _sc_gather_x(tids, x)
    y = _mlp(xb.reshape(E, CAP, D), W_gate, W_up, W_down)
    shared_out = _shared(x, shared_gate, shared_up, shared_down)
    out = _sc_combine(y.reshape(ECAP, D), slot8.reshape(NW, _SCC_IT, 16),
                      w8.reshape(NW, _SCC_IT, 16), shared_out)
    return out.reshape(B, S, Dm)


# speedup vs baseline: 1.0009x; 1.0009x over previous
import functools

import jax
import jax.numpy as jnp
from jax import lax
from jax.experimental import pallas as pl
from jax.experimental.pallas import tpu as pltpu
from jax.experimental.pallas import tpu_sc as plsc

E = 16
K = 8
D = 2048
I = 1024
N_GROUP = 4
GSZ = E // N_GROUP
TOPK_GROUP = 2
ROUTED_SCALING = 2.5
CAP = 640
ECAP = E * CAP
TRASH = ECAP
TIDS_N = ECAP + 8
T = 8192

NC, NS, L = 2, 16, 16
NW = NC * NS

INTERP = False

_NEG = -1e30


_BT = 512


def _router_body(x_ref, rw_ref, eb_ref, slot_ref, w_ref, cnt_ref):
    bt = _BT
    x = x_ref[...]
    logits = lax.dot_general(x, rw_ref[...], (((1,), (1,)), ((), ())),
                             preferred_element_type=jnp.float32)
    scores = jax.nn.sigmoid(logits)
    sfc = scores + eb_ref[...]

    li = lax.broadcasted_iota(jnp.int32, (bt, E), 1)
    lg = li // GSZ

    gs = []
    for g in range(N_GROUP):
        mg = lg == g
        vals = jnp.where(mg, sfc, _NEG)
        m1 = jnp.max(vals, axis=1, keepdims=True)
        pos1 = jnp.min(jnp.where(vals == m1, li, 99), axis=1, keepdims=True)
        m2 = jnp.max(jnp.where(li == pos1, _NEG, vals), axis=1, keepdims=True)
        gs.append(m1 + m2)

    best1 = gs[0]
    gi1 = jnp.zeros_like(gs[0], dtype=jnp.int32)
    for g in range(1, N_GROUP):
        b = gs[g] > best1
        best1 = jnp.where(b, gs[g], best1)
        gi1 = jnp.where(b, g, gi1)
    best2 = jnp.full_like(best1, _NEG)
    gi2 = jnp.full_like(gi1, -1)
    for g in range(N_GROUP):
        b = (gi1 != g) & (gs[g] > best2)
        best2 = jnp.where(b, gs[g], best2)
        gi2 = jnp.where(b, g, gi2)
    gmask = (lg == gi1) | (lg == gi2)

    masked = jnp.where(gmask, sfc, 0.0)

    cur = masked
    sel = jnp.zeros((bt, E), dtype=jnp.bool_)
    pos_list = []
    for _ in range(K):
        m = jnp.max(cur, axis=1, keepdims=True)
        pos = jnp.min(jnp.where(cur == m, li, 99), axis=1, keepdims=True)
        pos_list.append(pos)
        hit = li == pos
        sel = sel | hit
        cur = jnp.where(hit, _NEG, cur)

    selr = sel.astype(jnp.float32)
    wsum = jnp.sum(jnp.where(sel, scores, 0.0), axis=1, keepdims=True)
    inv = ROUTED_SCALING / (wsum + 1e-20)

    @pl.when(pl.program_id(0) == 0)
    def _():
        cnt_ref[...] = jnp.zeros_like(cnt_ref)

    r0 = lax.broadcasted_iota(jnp.int32, (bt, bt), 0)
    r1 = lax.broadcasted_iota(jnp.int32, (bt, bt), 1)
    tri = (r0 > r1).astype(jnp.float32)
    prefix = jnp.dot(tri, selr, preferred_element_type=jnp.float32)
    rank_f = prefix + cnt_ref[...]
    cnt_ref[...] = cnt_ref[...] + jnp.sum(selr, axis=0, keepdims=True)

    slot_cols = []
    w_cols = []
    for pos in pos_list:
        hit = li == pos
        rank_j = jnp.sum(jnp.where(hit, rank_f, 0.0), axis=1, keepdims=True)
        score_j = jnp.sum(jnp.where(hit, scores, 0.0), axis=1, keepdims=True)
        keep_j = rank_j < CAP
        w_cols.append(jnp.where(keep_j, score_j * inv, 0.0))
        slot_cols.append(jnp.where(keep_j, pos * CAP + rank_j.astype(jnp.int32),
                                   TRASH))
    slot_ref[...] = jnp.concatenate(slot_cols, axis=1)
    w_ref[...] = jnp.concatenate(w_cols, axis=1)


def _router(x, router_weight, e_bias):
    nblk = T // _BT
    return pl.pallas_call(
        _router_body,
        grid=(nblk,),
        in_specs=[
            pl.BlockSpec((_BT, D), lambda i: (i, 0)),
            pl.BlockSpec((E, D), lambda i: (0, 0)),
            pl.BlockSpec((1, E), lambda i: (0, 0)),
        ],
        out_specs=[
            pl.BlockSpec((_BT, K), lambda i: (i, 0)),
            pl.BlockSpec((_BT, K), lambda i: (i, 0)),
        ],
        out_shape=[
            jax.ShapeDtypeStruct((T, K), jnp.int32),
            jax.ShapeDtypeStruct((T, K), jnp.float32),
        ],
        scratch_shapes=[pltpu.VMEM((1, E), jnp.float32)],
        interpret=INTERP,
    )(x, router_weight, e_bias.reshape(1, E))


_SCA_CH = (T * K) // NW // 128

_sc_mesh = functools.partial(
    plsc.VectorSubcoreMesh, core_axis_name="c", subcore_axis_name="s")


def _sc_scatter_tids(slot_r, tval_r):
    @functools.partial(
        pl.kernel,
        mesh=_sc_mesh(),
        out_type=jax.ShapeDtypeStruct((TIDS_N,), jnp.int32),
        scratch_types=[
            pltpu.VMEM((_SCA_CH, 128), jnp.int32),
            pltpu.VMEM((_SCA_CH, 128), jnp.int32),
            pltpu.SemaphoreType.DMA,
        ],
    )
    def sca(slot_hbm, tval_hbm, tids_hbm, idx_v, val_v, sem):
        wid = lax.axis_index("s") * NC + lax.axis_index("c")
        pltpu.sync_copy(slot_hbm.at[wid], idx_v)
        pltpu.sync_copy(tval_hbm.at[wid], val_v)
        for j in range(_SCA_CH):
            pltpu.async_copy(val_v.at[j], tids_hbm.at[idx_v.at[j]], sem).wait()

    return sca(slot_r, tval_r)


_SCB_RPW = ECAP // NW
_SCB_CH = 16


def _sc_gather_x(tids, x):
    NCH = _SCB_RPW // _SCB_CH

    @functools.partial(
        pl.kernel,
        mesh=_sc_mesh(),
        out_type=jax.ShapeDtypeStruct((ECAP, D), jnp.float32),
        scratch_types=[
            pltpu.VMEM((_SCB_RPW,), jnp.int32),
            pltpu.VMEM((_SCB_CH, D), jnp.float32),
            pltpu.VMEM((_SCB_CH, D), jnp.float32),
            pltpu.SemaphoreType.DMA,
            pltpu.SemaphoreType.DMA,
        ],
    )
    def scb(tids_hbm, x_hbm, xb_hbm, idx_v, rows0, rows1, gsem, osem):
        wid = lax.axis_index("s") * NC + lax.axis_index("c")
        base = wid * _SCB_RPW
        bufs = (rows0, rows1)

        pltpu.sync_copy(tids_hbm.at[pl.ds(base, _SCB_RPW)],
                        idx_v.at[pl.ds(0, _SCB_RPW)])
        for j in range(NCH):
            v = idx_v[pl.ds(j * _SCB_CH, _SCB_CH)]
            idx_v[pl.ds(j * _SCB_CH, _SCB_CH)] = (
                jnp.minimum(jnp.maximum(v, 0), T - 1))

        def gref(c, buf):
            return pltpu.make_async_copy(
                x_hbm.at[idx_v.at[pl.ds(c * _SCB_CH, _SCB_CH)]], buf, gsem)

        def oref(c, buf):
            b = pl.multiple_of(base + c * _SCB_CH, _SCB_CH)
            return pltpu.make_async_copy(buf, xb_hbm.at[pl.ds(b, _SCB_CH)],
                                         osem)

        gref(0, rows0).start()
        gref(1, rows1).start()

        def body(j, _):
            for b in range(2):
                c = 2 * j + b
                gref(c, bufs[b]).wait()
                oref(c, bufs[b]).start()
                oref(c, bufs[b]).wait()

                @pl.when(c + 2 < NCH)
                def _(c=c, b=b):
                    gref(c + 2, bufs[b]).start()
            return 0

        lax.fori_loop(0, NCH // 2, body, 0)

    return scb(tids, x)


_NI = 2
_BI = I // _NI


def _mlp_body(xb_ref, wg_ref, wu_ref, wd_ref, y_ref):
    xb = xb_ref[0]
    g = jnp.dot(xb, wg_ref[0], preferred_element_type=jnp.float32)
    u = jnp.dot(xb, wu_ref[0], preferred_element_type=jnp.float32)
    h = g * jax.nn.sigmoid(g) * u
    part = jnp.dot(h, wd_ref[0], preferred_element_type=jnp.float32)

    @pl.when(pl.program_id(1) == 0)
    def _():
        y_ref[0] = part

    @pl.when(pl.program_id(1) != 0)
    def _():
        y_ref[0] = y_ref[0] + part


def _mlp(xb, W_gate, W_up, W_down):
    return pl.pallas_call(
        _mlp_body,
        grid=(E, _NI),
        in_specs=[
            pl.BlockSpec((1, CAP, D), lambda e, s: (e, 0, 0)),
            pl.BlockSpec((1, D, _BI), lambda e, s: (e, 0, s)),
            pl.BlockSpec((1, D, _BI), lambda e, s: (e, 0, s)),
            pl.BlockSpec((1, _BI, D), lambda e, s: (e, s, 0)),
        ],
        out_specs=pl.BlockSpec((1, CAP, D), lambda e, s: (e, 0, 0)),
        out_shape=jax.ShapeDtypeStruct((E, CAP, D), jnp.float32),
        interpret=INTERP,
    )(xb, W_gate, W_up, W_down)


_BT4 = 256


def _shared_body(x_ref, sg_ref, su_ref, sd_ref, o_ref):
    x = x_ref[...]
    g = lax.dot_general(x, sg_ref[...], (((1,), (1,)), ((), ())),
                        preferred_element_type=jnp.float32)
    u = lax.dot_general(x, su_ref[...], (((1,), (1,)), ((), ())),
                        preferred_element_type=jnp.float32)
    h = g * jax.nn.sigmoid(g) * u
    o_ref[...] = lax.dot_general(h, sd_ref[...], (((1,), (1,)), ((), ())),
                                 preferred_element_type=jnp.float32)


def _shared(x, shared_gate, shared_up, shared_down):
    SH = shared_gate.shape[0]
    return pl.pallas_call(
        _shared_body,
        grid=(T // _BT4,),
        in_specs=[
            pl.BlockSpec((_BT4, D), lambda i: (i, 0)),
            pl.BlockSpec((SH, D), lambda i: (0, 0)),
            pl.BlockSpec((SH, D), lambda i: (0, 0)),
            pl.BlockSpec((D, SH), lambda i: (0, 0)),
        ],
        out_specs=pl.BlockSpec((_BT4, D), lambda i: (i, 0)),
        out_shape=jax.ShapeDtypeStruct((T, D), jnp.float32),
        interpret=INTERP,
    )(x, shared_gate, shared_up, shared_down)


_SCC_TPW = T // NW
_SCC_IT = _SCC_TPW // 2


def _sc_combine(y, slot_r, w_r, shared_out):
    NCH = _SCC_IT

    @functools.partial(
        pl.kernel,
        mesh=_sc_mesh(),
        out_type=jax.ShapeDtypeStruct((T, D), jnp.float32),
        scratch_types=[
            pltpu.VMEM((16,), jnp.int32),
            pltpu.VMEM((16,), jnp.int32),
            pltpu.VMEM((16,), jnp.float32),
            pltpu.VMEM((16,), jnp.float32),
            pltpu.VMEM((16, D), jnp.float32),
            pltpu.VMEM((16, D), jnp.float32),
            pltpu.VMEM((2, D), jnp.float32),
            pltpu.VMEM((2, D), jnp.float32),
            pltpu.VMEM((2, D), jnp.float32),
            pltpu.VMEM((2, D), jnp.float32),
            pltpu.SemaphoreType.DMA,
            pltpu.SemaphoreType.DMA,
            pltpu.SemaphoreType.DMA,
            pltpu.SemaphoreType.DMA,
        ],
    )
    def scc(y_hbm, slot_hbm, w_hbm, sh_hbm, out_hbm, idx0, idx1, wv0, wv1,
            rows0, rows1, acc0, acc1, acc2, acc3, asem, gsem, ssem, osem):
        wid = lax.axis_index("s") * NC + lax.axis_index("c")
        tbase = wid * _SCC_TPW
        idxs = (idx0, idx1)
        wvs = (wv0, wv1)
        rows = (rows0, rows1)
        accs = (acc0, acc1, acc2, acc3)
        _dn = lax.GatherDimensionNumbers(offset_dims=(),
                                         collapsed_slice_dims=(0,),
                                         start_index_map=(0,))

        def iref(c, p):
            return pltpu.make_async_copy(slot_hbm.at[wid, c], idxs[p], asem)

        def wref(c, p):
            return pltpu.make_async_copy(w_hbm.at[wid, c], wvs[p], asem)

        def gref(p):
            return pltpu.make_async_copy(y_hbm.at[idxs[p]], rows[p], gsem)

        def sref(c, a):
            t0 = pl.multiple_of(tbase + 2 * c, 2)
            return pltpu.make_async_copy(sh_hbm.at[pl.ds(t0, 2)], accs[a],
                                         ssem)

        def oref(c, a):
            t0 = pl.multiple_of(tbase + 2 * c, 2)
            return pltpu.make_async_copy(accs[a], out_hbm.at[pl.ds(t0, 2)],
                                         osem)

        def clamp(p):
            v = idxs[p][...]
            idxs[p][...] = jnp.minimum(jnp.maximum(v, 0), ECAP - 1)

        for b in range(2):
            iref(b, b).start()
            wref(b, b).start()
        for b in range(2):
            iref(b, b).wait()
            wref(b, b).wait()
            clamp(b)
            gref(b).start()
            sref(b, b).start()

        def body(j, _):
            for b in range(4):
                c = 4 * j + b
                p = b % 2
                gref(p).wait()
                w16 = wvs[p][...]

                def smalls(c=c, p=p):
                    iref(c + 2, p).start()
                    wref(c + 2, p).start()

                if b < 2:
                    smalls()
                else:
                    pl.when(j < (NCH // 4) - 1)(smalls)
                sref(c, b).wait()
                wsp = [lax.gather(w16, jnp.full((L, 1), jj, jnp.int32),
                                  _dn, (1,),
                                  mode=lax.GatherScatterMode.PROMISE_IN_BOUNDS)
                       for jj in range(16)]

                for a in range(2):
                    def cbody(cc, _, a=a):
                        sl = pl.ds(pl.multiple_of(cc * L, L), L)
                        acc = accs[b][a, sl]
                        for jj in range(K):
                            acc = acc + wsp[a * K + jj] * rows[p][a * K + jj,
                                                                  sl]
                        accs[b][a, sl] = acc
                        return 0

                    lax.fori_loop(0, D // L, cbody, 0)
                oref(c, b).start()

                def owait(c=c, b=b):
                    oref(c - 2, (b + 2) % 4).wait()

                if b >= 2:
                    owait()
                else:
                    pl.when(j > 0)(owait)

                def nxt(c=c, p=p, b=b):
                    iref(c + 2, p).wait()
                    wref(c + 2, p).wait()
                    clamp(p)
                    gref(p).start()
                    sref(c + 2, (b + 2) % 4).start()

                if b < 2:
                    nxt()
                else:
                    pl.when(j < (NCH // 4) - 1)(nxt)
            return 0

        lax.fori_loop(0, NCH // 4, body, 0)
        oref(NCH - 2, 2).wait()
        oref(NCH - 1, 3).wait()

    return scc(y, slot_r, w_r, shared_out)


def kernel(hidden_states, router_weight, e_bias, W_gate, W_up, W_down,
           shared_gate, shared_up, shared_down):
    B, S, Dm = hidden_states.shape
    x = hidden_states.reshape(T, Dm)

    slot8, w8 = _router(x, router_weight, e_bias)

    tvals = jnp.broadcast_to(jnp.arange(T, dtype=jnp.int32)[:, None], (T, K))
    tids = _sc_scatter_tids(slot8.reshape(NW, _SCA_CH, 128),
                            tvals.reshape(NW, _SCA_CH, 128))

    xb = _sc_gather_x(tids, x)
    y = _mlp(xb.reshape(E, CAP, D), W_gate, W_up, W_down)
    shared_out = _shared(x, shared_gate, shared_up, shared_down)
    out = _sc_combine(y.reshape(ECAP, D), slot8.reshape(NW, _SCC_IT, 16),
                      w8.reshape(NW, _SCC_IT, 16), shared_out)
    return out.reshape(B, S, Dm)


# speedup vs baseline: 6.7306x; 6.7242x over previous
import jax
import jax.numpy as jnp
from jax import lax
from jax.experimental import pallas as pl
from jax.experimental.pallas import tpu as pltpu

E = 16
K = 8
D = 2048
I = 1024
N_GROUP = 4
GSZ = E // N_GROUP
ROUTED_SCALING = 2.5
CAP = 640
ECAP = E * CAP
T = 8192

INTERP = False

_NEG = -1e30


_BT = 512


def _router_body(x_ref, rw_ref, eb_ref, w_ref, rk_ref, cnt_ref):
    bt = _BT
    x = x_ref[...]
    logits = lax.dot_general(x, rw_ref[...], (((1,), (1,)), ((), ())),
                             preferred_element_type=jnp.float32)
    scores = jax.nn.sigmoid(logits)
    sfc = scores + eb_ref[...]

    li = lax.broadcasted_iota(jnp.int32, (bt, E), 1)
    lg = li // GSZ

    gs = []
    for g in range(N_GROUP):
        vals = jnp.where(lg == g, sfc, _NEG)
        m1 = jnp.max(vals, axis=1, keepdims=True)
        pos1 = jnp.min(jnp.where(vals == m1, li, 99), axis=1, keepdims=True)
        m2 = jnp.max(jnp.where(li == pos1, _NEG, vals), axis=1, keepdims=True)
        gs.append(m1 + m2)

    best1 = gs[0]
    gi1 = jnp.zeros_like(gs[0], dtype=jnp.int32)
    for g in range(1, N_GROUP):
        b = gs[g] > best1
        best1 = jnp.where(b, gs[g], best1)
        gi1 = jnp.where(b, g, gi1)
    best2 = jnp.full_like(best1, _NEG)
    gi2 = jnp.full_like(gi1, -1)
    for g in range(N_GROUP):
        b = (gi1 != g) & (gs[g] > best2)
        best2 = jnp.where(b, gs[g], best2)
        gi2 = jnp.where(b, g, gi2)
    gmask = (lg == gi1) | (lg == gi2)

    masked = jnp.where(gmask, sfc, 0.0)

    cur = masked
    sel = jnp.zeros((bt, E), dtype=jnp.bool_)
    for _ in range(K):
        m = jnp.max(cur, axis=1, keepdims=True)
        pos = jnp.min(jnp.where(cur == m, li, 99), axis=1, keepdims=True)
        hit = li == pos
        sel = sel | hit
        cur = jnp.where(hit, _NEG, cur)

    selr = sel.astype(jnp.float32)
    wsum = jnp.sum(jnp.where(sel, scores, 0.0), axis=1, keepdims=True)
    inv = ROUTED_SCALING / (wsum + 1e-20)

    @pl.when(pl.program_id(0) == 0)
    def _():
        cnt_ref[...] = jnp.zeros_like(cnt_ref)

    r0 = lax.broadcasted_iota(jnp.int32, (bt, bt), 0)
    r1 = lax.broadcasted_iota(jnp.int32, (bt, bt), 1)
    tri = (r0 > r1).astype(jnp.float32)
    prefix = jnp.dot(tri, selr, preferred_element_type=jnp.float32)
    rank_f = prefix + cnt_ref[...]
    cnt_ref[...] = cnt_ref[...] + jnp.sum(selr, axis=0, keepdims=True)

    keep = sel & (rank_f < CAP)
    w_ref[...] = jnp.where(keep, scores * inv, 0.0)
    rk_ref[...] = jnp.where(sel, rank_f, -1.0)


def _router(x, router_weight, e_bias):
    return pl.pallas_call(
        _router_body,
        grid=(T // _BT,),
        in_specs=[
            pl.BlockSpec((_BT, D), lambda i: (i, 0)),
            pl.BlockSpec((E, D), lambda i: (0, 0)),
            pl.BlockSpec((1, E), lambda i: (0, 0)),
        ],
        out_specs=[
            pl.BlockSpec((_BT, E), lambda i: (i, 0)),
            pl.BlockSpec((_BT, E), lambda i: (i, 0)),
        ],
        out_shape=[
            jax.ShapeDtypeStruct((T, E), jnp.float32),
            jax.ShapeDtypeStruct((T, E), jnp.float32),
        ],
        scratch_shapes=[pltpu.VMEM((1, E), jnp.float32)],
        interpret=INTERP,
    )(x, router_weight, e_bias.reshape(1, E))


_EG = 4
_NG = E // _EG
_BTD = 1024
_NTB = T // _BTD


def _dispatch_body(rk_ref, x_ref, xb_ref):
    x = x_ref[...]
    ir = lax.broadcasted_iota(jnp.int32, (_BTD, CAP), 1)
    li = lax.broadcasted_iota(jnp.int32, (_BTD, E), 1)
    rk = rk_ref[...]
    for eg in range(_EG):
        eidx = pl.program_id(0) * _EG + eg
        col = jnp.sum(jnp.where(li == eidx, rk, 0.0), axis=1,
                      keepdims=True).astype(jnp.int32)
        m = (col == ir).astype(jnp.bfloat16)
        part = lax.dot_general(m, x, (((0,), (0,)), ((), ())),
                               preferred_element_type=jnp.float32)
        sl = pl.ds(eg * CAP, CAP)

        @pl.when(pl.program_id(1) == 0)
        def _():
            xb_ref[sl, :] = part.astype(jnp.bfloat16)

        @pl.when(pl.program_id(1) != 0)
        def _():
            xb_ref[sl, :] = xb_ref[sl, :] + part.astype(jnp.bfloat16)


def _dispatch(x_bf, rankv):
    return pl.pallas_call(
        _dispatch_body,
        grid=(_NG, _NTB),
        in_specs=[
            pl.BlockSpec((_BTD, E), lambda g, t: (t, 0)),
            pl.BlockSpec((_BTD, D), lambda g, t: (t, 0)),
        ],
        out_specs=pl.BlockSpec((_EG * CAP, D), lambda g, t: (g, 0)),
        out_shape=jax.ShapeDtypeStruct((ECAP, D), jnp.bfloat16),
        interpret=INTERP,
    )(rankv, x_bf)


def _mlp_body(xb_ref, wg_ref, wu_ref, wd_ref, y_ref):
    xb = xb_ref[0]
    g = jnp.dot(xb, wg_ref[0], preferred_element_type=jnp.float32)
    u = jnp.dot(xb, wu_ref[0], preferred_element_type=jnp.float32)
    h = (g * jax.nn.sigmoid(g) * u).astype(jnp.bfloat16)
    y_ref[0] = jnp.dot(h, wd_ref[0],
                       preferred_element_type=jnp.float32).astype(jnp.bfloat16)


def _mlp(xb, W_gate, W_up, W_down):
    return pl.pallas_call(
        _mlp_body,
        grid=(E,),
        in_specs=[
            pl.BlockSpec((1, CAP, D), lambda e: (e, 0, 0)),
            pl.BlockSpec((1, D, I), lambda e: (e, 0, 0)),
            pl.BlockSpec((1, D, I), lambda e: (e, 0, 0)),
            pl.BlockSpec((1, I, D), lambda e: (e, 0, 0)),
        ],
        out_specs=pl.BlockSpec((1, CAP, D), lambda e: (e, 0, 0)),
        out_shape=jax.ShapeDtypeStruct((E, CAP, D), jnp.bfloat16),
        interpret=INTERP,
    )(xb, W_gate, W_up, W_down)


_BT4 = 512


def _shared_body(x_ref, sg_ref, su_ref, sd_ref, o_ref):
    x = x_ref[...]
    g = lax.dot_general(x, sg_ref[...], (((1,), (1,)), ((), ())),
                        preferred_element_type=jnp.float32)
    u = lax.dot_general(x, su_ref[...], (((1,), (1,)), ((), ())),
                        preferred_element_type=jnp.float32)
    h = (g * jax.nn.sigmoid(g) * u).astype(jnp.bfloat16)
    o_ref[...] = lax.dot_general(h, sd_ref[...], (((1,), (1,)), ((), ())),
                                 preferred_element_type=jnp.float32)


def _shared(x_bf, shared_gate, shared_up, shared_down):
    SH = shared_gate.shape[0]
    return pl.pallas_call(
        _shared_body,
        grid=(T // _BT4,),
        in_specs=[
            pl.BlockSpec((_BT4, D), lambda i: (i, 0)),
            pl.BlockSpec((SH, D), lambda i: (0, 0)),
            pl.BlockSpec((SH, D), lambda i: (0, 0)),
            pl.BlockSpec((D, SH), lambda i: (0, 0)),
        ],
        out_specs=pl.BlockSpec((_BT4, D), lambda i: (i, 0)),
        out_shape=jax.ShapeDtypeStruct((T, D), jnp.float32),
        interpret=INTERP,
    )(x_bf, shared_gate, shared_up, shared_down)


_BTC = 1024


def _combine_body(rk_ref, w_ref, y_ref, sh_ref, o_ref):
    ir = lax.broadcasted_iota(jnp.int32, (_BTC, CAP), 1)
    li = lax.broadcasted_iota(jnp.int32, (_BTC, E), 1)
    eidx = pl.program_id(1)
    col = jnp.sum(jnp.where(li == eidx, rk_ref[...], 0.0), axis=1,
                  keepdims=True).astype(jnp.int32)
    wcol = jnp.sum(jnp.where(li == eidx, w_ref[...], 0.0), axis=1,
                   keepdims=True)
    a = (col == ir).astype(jnp.bfloat16)
    gath = jnp.dot(a, y_ref[0], preferred_element_type=jnp.float32)
    contrib = wcol * gath

    @pl.when(pl.program_id(1) == 0)
    def _():
        o_ref[...] = sh_ref[...] + contrib

    @pl.when(pl.program_id(1) != 0)
    def _():
        o_ref[...] = o_ref[...] + contrib


def _combine(y, rankv, wfull, shared_out):
    return pl.pallas_call(
        _combine_body,
        grid=(T // _BTC, E),
        in_specs=[
            pl.BlockSpec((_BTC, E), lambda t, e: (t, 0)),
            pl.BlockSpec((_BTC, E), lambda t, e: (t, 0)),
            pl.BlockSpec((1, CAP, D), lambda t, e: (e, 0, 0)),
            pl.BlockSpec((_BTC, D), lambda t, e: (t, 0)),
        ],
        out_specs=pl.BlockSpec((_BTC, D), lambda t, e: (t, 0)),
        out_shape=jax.ShapeDtypeStruct((T, D), jnp.float32),
        interpret=INTERP,
    )(rankv, wfull, y, shared_out)


def kernel(hidden_states, router_weight, e_bias, W_gate, W_up, W_down,
           shared_gate, shared_up, shared_down):
    B, S, Dm = hidden_states.shape
    x = hidden_states.reshape(T, Dm)
    x_bf = x.astype(jnp.bfloat16)

    wfull, rankv = _router(x, router_weight, e_bias)
    xb = _dispatch(x_bf, rankv)
    y = _mlp(xb.reshape(E, CAP, D),
             W_gate.astype(jnp.bfloat16),
             W_up.astype(jnp.bfloat16),
             W_down.astype(jnp.bfloat16))
    shared_out = _shared(x_bf,
                         shared_gate.astype(jnp.bfloat16),
                         shared_up.astype(jnp.bfloat16),
                         shared_down.astype(jnp.bfloat16))
    out = _combine(y, rankv, wfull, shared_out)
    return out.reshape(B, S, Dm)


# speedup vs baseline: 9.9604x; 1.4799x over previous
import jax
import jax.numpy as jnp
from jax import lax
from jax.experimental import pallas as pl
from jax.experimental.pallas import tpu as pltpu

E = 16
K = 8
D = 2048
I = 1024
N_GROUP = 4
GSZ = E // N_GROUP
ROUTED_SCALING = 2.5
CAP = 640
ECAP = E * CAP
T = 8192

INTERP = False

_NEG = -1e30


_BT = 512


def _router_body(x_ref, rw_ref, eb_ref, w_ref, rk_ref, cnt_ref):
    bt = _BT
    x = x_ref[...]
    logits = lax.dot_general(x, rw_ref[...], (((1,), (1,)), ((), ())),
                             preferred_element_type=jnp.float32)
    scores = jax.nn.sigmoid(logits)
    sfc = scores + eb_ref[...]

    li = lax.broadcasted_iota(jnp.int32, (bt, E), 1)
    lg = li // GSZ

    gs = []
    for g in range(N_GROUP):
        vals = jnp.where(lg == g, sfc, _NEG)
        m1 = jnp.max(vals, axis=1, keepdims=True)
        pos1 = jnp.min(jnp.where(vals == m1, li, 99), axis=1, keepdims=True)
        m2 = jnp.max(jnp.where(li == pos1, _NEG, vals), axis=1, keepdims=True)
        gs.append(m1 + m2)

    best1 = gs[0]
    gi1 = jnp.zeros_like(gs[0], dtype=jnp.int32)
    for g in range(1, N_GROUP):
        b = gs[g] > best1
        best1 = jnp.where(b, gs[g], best1)
        gi1 = jnp.where(b, g, gi1)
    best2 = jnp.full_like(best1, _NEG)
    gi2 = jnp.full_like(gi1, -1)
    for g in range(N_GROUP):
        b = (gi1 != g) & (gs[g] > best2)
        best2 = jnp.where(b, gs[g], best2)
        gi2 = jnp.where(b, g, gi2)
    gmask = (lg == gi1) | (lg == gi2)

    masked = jnp.where(gmask, sfc, 0.0)

    cur = masked
    sel = jnp.zeros((bt, E), dtype=jnp.bool_)
    for _ in range(K):
        m = jnp.max(cur, axis=1, keepdims=True)
        pos = jnp.min(jnp.where(cur == m, li, 99), axis=1, keepdims=True)
        hit = li == pos
        sel = sel | hit
        cur = jnp.where(hit, _NEG, cur)

    selr = sel.astype(jnp.float32)
    wsum = jnp.sum(jnp.where(sel, scores, 0.0), axis=1, keepdims=True)
    inv = ROUTED_SCALING / (wsum + 1e-20)

    @pl.when(pl.program_id(0) == 0)
    def _():
        cnt_ref[...] = jnp.zeros_like(cnt_ref)

    r0 = lax.broadcasted_iota(jnp.int32, (bt, bt), 0)
    r1 = lax.broadcasted_iota(jnp.int32, (bt, bt), 1)
    tri = (r0 > r1).astype(jnp.bfloat16)
    prefix = jnp.dot(tri, selr.astype(jnp.bfloat16),
                     preferred_element_type=jnp.float32)
    rank_f = prefix + cnt_ref[...]
    cnt_ref[...] = cnt_ref[...] + jnp.sum(selr, axis=0, keepdims=True)

    keep = sel & (rank_f < CAP)
    w_ref[...] = jnp.where(keep, scores * inv, 0.0)
    rk_ref[...] = jnp.where(sel, rank_f, -1.0)


def _router(x, router_weight, e_bias):
    return pl.pallas_call(
        _router_body,
        grid=(T // _BT,),
        in_specs=[
            pl.BlockSpec((_BT, D), lambda i: (i, 0)),
            pl.BlockSpec((E, D), lambda i: (0, 0)),
            pl.BlockSpec((1, E), lambda i: (0, 0)),
        ],
        out_specs=[
            pl.BlockSpec((_BT, E), lambda i: (i, 0)),
            pl.BlockSpec((_BT, E), lambda i: (i, 0)),
        ],
        out_shape=[
            jax.ShapeDtypeStruct((T, E), jnp.float32),
            jax.ShapeDtypeStruct((T, E), jnp.float32),
        ],
        scratch_shapes=[pltpu.VMEM((1, E), jnp.float32)],
        interpret=INTERP,
    )(x, router_weight, e_bias.reshape(1, E))


_EG = 4
_NG = E // _EG
_BTD = 1024
_NTB = T // _BTD


def _dispatch_body(rk_ref, x_ref, xb_ref):
    x = x_ref[...]
    ir = lax.broadcasted_iota(jnp.int32, (_BTD, CAP), 1)
    li = lax.broadcasted_iota(jnp.int32, (_BTD, E), 1)
    rk = rk_ref[...]
    for eg in range(_EG):
        eidx = pl.program_id(0) * _EG + eg
        col = jnp.sum(jnp.where(li == eidx, rk, 0.0), axis=1,
                      keepdims=True).astype(jnp.int32)
        sl = pl.ds(eg * CAP, CAP)

        @pl.when(pl.program_id(1) == 0)
        def _():
            xb_ref[sl, :] = jnp.zeros((CAP, D), jnp.bfloat16)

        @pl.when(jnp.any((col >= 0) & (col < CAP)))
        def _(col=col):
            m = (col == ir).astype(jnp.bfloat16)
            part = lax.dot_general(m, x, (((0,), (0,)), ((), ())),
                                   preferred_element_type=jnp.float32)
            xb_ref[sl, :] = xb_ref[sl, :] + part.astype(jnp.bfloat16)


def _dispatch(x_bf, rankv):
    return pl.pallas_call(
        _dispatch_body,
        grid=(_NG, _NTB),
        in_specs=[
            pl.BlockSpec((_BTD, E), lambda g, t: (t, 0)),
            pl.BlockSpec((_BTD, D), lambda g, t: (t, 0)),
        ],
        out_specs=pl.BlockSpec((_EG * CAP, D), lambda g, t: (g, 0)),
        out_shape=jax.ShapeDtypeStruct((ECAP, D), jnp.bfloat16),
        interpret=INTERP,
    )(rankv, x_bf)


def _mlp_body(xb_ref, wg_ref, wu_ref, wd_ref, y_ref):
    xb = xb_ref[0]
    g = jnp.dot(xb, wg_ref[0], preferred_element_type=jnp.float32)
    u = jnp.dot(xb, wu_ref[0], preferred_element_type=jnp.float32)
    h = (g * jax.nn.sigmoid(g) * u).astype(jnp.bfloat16)
    y_ref[0] = jnp.dot(h, wd_ref[0],
                       preferred_element_type=jnp.float32).astype(jnp.bfloat16)


def _mlp(xb, W_gate, W_up, W_down):
    return pl.pallas_call(
        _mlp_body,
        grid=(E,),
        in_specs=[
            pl.BlockSpec((1, CAP, D), lambda e: (e, 0, 0)),
            pl.BlockSpec((1, D, I), lambda e: (e, 0, 0)),
            pl.BlockSpec((1, D, I), lambda e: (e, 0, 0)),
            pl.BlockSpec((1, I, D), lambda e: (e, 0, 0)),
        ],
        out_specs=pl.BlockSpec((1, CAP, D), lambda e: (e, 0, 0)),
        out_shape=jax.ShapeDtypeStruct((E, CAP, D), jnp.bfloat16),
        interpret=INTERP,
    )(xb, W_gate, W_up, W_down)


_BT4 = 512


def _shared_body(x_ref, sg_ref, su_ref, sd_ref, o_ref):
    x = x_ref[...]
    g = lax.dot_general(x, sg_ref[...], (((1,), (1,)), ((), ())),
                        preferred_element_type=jnp.float32)
    u = lax.dot_general(x, su_ref[...], (((1,), (1,)), ((), ())),
                        preferred_element_type=jnp.float32)
    h = (g * jax.nn.sigmoid(g) * u).astype(jnp.bfloat16)
    o_ref[...] = lax.dot_general(h, sd_ref[...], (((1,), (1,)), ((), ())),
                                 preferred_element_type=jnp.float32)


def _shared(x_bf, shared_gate, shared_up, shared_down):
    SH = shared_gate.shape[0]
    return pl.pallas_call(
        _shared_body,
        grid=(T // _BT4,),
        in_specs=[
            pl.BlockSpec((_BT4, D), lambda i: (i, 0)),
            pl.BlockSpec((SH, D), lambda i: (0, 0)),
            pl.BlockSpec((SH, D), lambda i: (0, 0)),
            pl.BlockSpec((D, SH), lambda i: (0, 0)),
        ],
        out_specs=pl.BlockSpec((_BT4, D), lambda i: (i, 0)),
        out_shape=jax.ShapeDtypeStruct((T, D), jnp.float32),
        interpret=INTERP,
    )(x_bf, shared_gate, shared_up, shared_down)


_BTC = 1024


def _combine_body(rk_ref, w_ref, y_ref, sh_ref, o_ref):
    ir = lax.broadcasted_iota(jnp.int32, (_BTC, CAP), 1)
    li = lax.broadcasted_iota(jnp.int32, (_BTC, E), 1)
    eidx = pl.program_id(1)
    col = jnp.sum(jnp.where(li == eidx, rk_ref[...], 0.0), axis=1,
                  keepdims=True).astype(jnp.int32)
    wcol = jnp.sum(jnp.where(li == eidx, w_ref[...], 0.0), axis=1,
                   keepdims=True)

    @pl.when(pl.program_id(1) == 0)
    def _():
        o_ref[...] = sh_ref[...]

    @pl.when(jnp.any((col >= 0) & (col < CAP)))
    def _():
        a = (col == ir).astype(jnp.bfloat16)
        gath = jnp.dot(a, y_ref[0], preferred_element_type=jnp.float32)
        o_ref[...] = o_ref[...] + wcol * gath


def _combine(y, rankv, wfull, shared_out):
    return pl.pallas_call(
        _combine_body,
        grid=(T // _BTC, E),
        in_specs=[
            pl.BlockSpec((_BTC, E), lambda t, e: (t, 0)),
            pl.BlockSpec((_BTC, E), lambda t, e: (t, 0)),
            pl.BlockSpec((1, CAP, D), lambda t, e: (e, 0, 0)),
            pl.BlockSpec((_BTC, D), lambda t, e: (t, 0)),
        ],
        out_specs=pl.BlockSpec((_BTC, D), lambda t, e: (t, 0)),
        out_shape=jax.ShapeDtypeStruct((T, D), jnp.float32),
        interpret=INTERP,
    )(rankv, wfull, y, shared_out)


def kernel(hidden_states, router_weight, e_bias, W_gate, W_up, W_down,
           shared_gate, shared_up, shared_down):
    B, S, Dm = hidden_states.shape
    x = hidden_states.reshape(T, Dm)
    x_bf = x.astype(jnp.bfloat16)

    wfull, rankv = _router(x, router_weight, e_bias)
    xb = _dispatch(x_bf, rankv)
    y = _mlp(xb.reshape(E, CAP, D),
             W_gate.astype(jnp.bfloat16),
             W_up.astype(jnp.bfloat16),
             W_down.astype(jnp.bfloat16))
    shared_out = _shared(x_bf,
                         shared_gate.astype(jnp.bfloat16),
                         shared_up.astype(jnp.bfloat16),
                         shared_down.astype(jnp.bfloat16))
    out = _combine(y, rankv, wfull, shared_out)
    return out.reshape(B, S, Dm)


# speedup vs baseline: 11.5522x; 1.1598x over previous
import jax
import jax.numpy as jnp
from jax import lax
from jax.experimental import pallas as pl
from jax.experimental.pallas import tpu as pltpu

E = 16
K = 8
D = 2048
I = 1024
N_GROUP = 4
GSZ = E // N_GROUP
ROUTED_SCALING = 2.5
CAP = 640
ECAP = E * CAP
T = 8192

INTERP = False

_NEG = -1e30


_BT = 512


def _router_body(x_ref, rw_ref, eb_ref, w_ref, rk_ref, cnt_ref):
    bt = _BT
    x = x_ref[...]
    logits = lax.dot_general(x, rw_ref[...], (((1,), (1,)), ((), ())),
                             preferred_element_type=jnp.float32)
    scores = jax.nn.sigmoid(logits)
    sfc = scores + eb_ref[...]

    li = lax.broadcasted_iota(jnp.int32, (bt, E), 1)
    lg = li // GSZ

    gs = []
    for g in range(N_GROUP):
        vals = jnp.where(lg == g, sfc, _NEG)
        m1 = jnp.max(vals, axis=1, keepdims=True)
        pos1 = jnp.min(jnp.where(vals == m1, li, 99), axis=1, keepdims=True)
        m2 = jnp.max(jnp.where(li == pos1, _NEG, vals), axis=1, keepdims=True)
        gs.append(m1 + m2)

    best1 = gs[0]
    gi1 = jnp.zeros_like(gs[0], dtype=jnp.int32)
    for g in range(1, N_GROUP):
        b = gs[g] > best1
        best1 = jnp.where(b, gs[g], best1)
        gi1 = jnp.where(b, g, gi1)
    best2 = jnp.full_like(best1, _NEG)
    gi2 = jnp.full_like(gi1, -1)
    for g in range(N_GROUP):
        b = (gi1 != g) & (gs[g] > best2)
        best2 = jnp.where(b, gs[g], best2)
        gi2 = jnp.where(b, g, gi2)
    gmask = (lg == gi1) | (lg == gi2)

    masked = jnp.where(gmask, sfc, 0.0)

    cur = masked
    sel = jnp.zeros((bt, E), dtype=jnp.bool_)
    for _ in range(K):
        m = jnp.max(cur, axis=1, keepdims=True)
        pos = jnp.min(jnp.where(cur == m, li, 99), axis=1, keepdims=True)
        hit = li == pos
        sel = sel | hit
        cur = jnp.where(hit, _NEG, cur)

    selr = sel.astype(jnp.float32)
    wsum = jnp.sum(jnp.where(sel, scores, 0.0), axis=1, keepdims=True)
    inv = ROUTED_SCALING / (wsum + 1e-20)

    @pl.when(pl.program_id(0) == 0)
    def _():
        cnt_ref[...] = jnp.zeros_like(cnt_ref)

    r0 = lax.broadcasted_iota(jnp.int32, (bt, bt), 0)
    r1 = lax.broadcasted_iota(jnp.int32, (bt, bt), 1)
    tri = (r0 > r1).astype(jnp.bfloat16)
    prefix = jnp.dot(tri, selr.astype(jnp.bfloat16),
                     preferred_element_type=jnp.float32)
    rank_f = prefix + cnt_ref[...]
    cnt_ref[...] = cnt_ref[...] + jnp.sum(selr, axis=0, keepdims=True)

    keep = sel & (rank_f < CAP)
    w_ref[...] = jnp.where(keep, scores * inv, 0.0)
    rk_ref[...] = jnp.where(sel, rank_f, -1.0)


def _router(x, router_weight, e_bias):
    return pl.pallas_call(
        _router_body,
        grid=(T // _BT,),
        in_specs=[
            pl.BlockSpec((_BT, D), lambda i: (i, 0)),
            pl.BlockSpec((E, D), lambda i: (0, 0)),
            pl.BlockSpec((1, E), lambda i: (0, 0)),
        ],
        out_specs=[
            pl.BlockSpec((_BT, E), lambda i: (i, 0)),
            pl.BlockSpec((_BT, E), lambda i: (i, 0)),
        ],
        out_shape=[
            jax.ShapeDtypeStruct((T, E), jnp.float32),
            jax.ShapeDtypeStruct((T, E), jnp.float32),
        ],
        scratch_shapes=[pltpu.VMEM((1, E), jnp.float32)],
        interpret=INTERP,
    )(x, router_weight, e_bias.reshape(1, E))


_EG = 4
_NG = E // _EG
_BTD = 1024
_NTB = T // _BTD


def _dispatch_body(rk_ref, x_ref, xb_ref):
    x = x_ref[...]
    ir = lax.broadcasted_iota(jnp.int32, (_BTD, CAP), 1)
    li = lax.broadcasted_iota(jnp.int32, (_BTD, E), 1)
    rk = rk_ref[...]
    for eg in range(_EG):
        eidx = pl.program_id(0) * _EG + eg
        col = jnp.sum(jnp.where(li == eidx, rk, 0.0), axis=1,
                      keepdims=True).astype(jnp.int32)
        sl = pl.ds(eg * CAP, CAP)

        @pl.when(pl.program_id(1) == 0)
        def _():
            xb_ref[sl, :] = jnp.zeros((CAP, D), jnp.bfloat16)

        @pl.when(jnp.any((col >= 0) & (col < CAP)))
        def _(col=col):
            m = (col == ir).astype(jnp.bfloat16)
            part = lax.dot_general(m, x, (((0,), (0,)), ((), ())),
                                   preferred_element_type=jnp.float32)
            xb_ref[sl, :] = xb_ref[sl, :] + part.astype(jnp.bfloat16)


def _dispatch(x_bf, rankv):
    return pl.pallas_call(
        _dispatch_body,
        grid=(_NG, _NTB),
        in_specs=[
            pl.BlockSpec((_BTD, E), lambda g, t: (t, 0)),
            pl.BlockSpec((_BTD, D), lambda g, t: (t, 0)),
        ],
        out_specs=pl.BlockSpec((_EG * CAP, D), lambda g, t: (g, 0)),
        out_shape=jax.ShapeDtypeStruct((ECAP, D), jnp.bfloat16),
        interpret=INTERP,
    )(rankv, x_bf)


_NI = 2
_BI = I // _NI


def _mlp_body(xb_ref, wg_ref, wu_ref, wd_ref, y_ref):
    xb = xb_ref[0]
    wg = wg_ref[0].astype(jnp.bfloat16)
    wu = wu_ref[0].astype(jnp.bfloat16)
    wd = wd_ref[0].astype(jnp.bfloat16)
    g = jnp.dot(xb, wg, preferred_element_type=jnp.float32)
    u = jnp.dot(xb, wu, preferred_element_type=jnp.float32)
    h = (g * jax.nn.sigmoid(g) * u).astype(jnp.bfloat16)
    part = jnp.dot(h, wd, preferred_element_type=jnp.float32)

    @pl.when(pl.program_id(1) == 0)
    def _():
        y_ref[0] = part.astype(jnp.bfloat16)

    @pl.when(pl.program_id(1) != 0)
    def _():
        y_ref[0] = (y_ref[0].astype(jnp.float32) + part).astype(jnp.bfloat16)


def _mlp(xb, W_gate, W_up, W_down):
    return pl.pallas_call(
        _mlp_body,
        grid=(E, _NI),
        in_specs=[
            pl.BlockSpec((1, CAP, D), lambda e, s: (e, 0, 0)),
            pl.BlockSpec((1, D, _BI), lambda e, s: (e, 0, s)),
            pl.BlockSpec((1, D, _BI), lambda e, s: (e, 0, s)),
            pl.BlockSpec((1, _BI, D), lambda e, s: (e, s, 0)),
        ],
        out_specs=pl.BlockSpec((1, CAP, D), lambda e, s: (e, 0, 0)),
        out_shape=jax.ShapeDtypeStruct((E, CAP, D), jnp.bfloat16),
        interpret=INTERP,
    )(xb, W_gate, W_up, W_down)


_BT4 = 512


def _shared_body(x_ref, sg_ref, su_ref, sd_ref, o_ref):
    x = x_ref[...]
    g = lax.dot_general(x, sg_ref[...], (((1,), (1,)), ((), ())),
                        preferred_element_type=jnp.float32)
    u = lax.dot_general(x, su_ref[...], (((1,), (1,)), ((), ())),
                        preferred_element_type=jnp.float32)
    h = (g * jax.nn.sigmoid(g) * u).astype(jnp.bfloat16)
    o_ref[...] = lax.dot_general(h, sd_ref[...], (((1,), (1,)), ((), ())),
                                 preferred_element_type=jnp.float32)


def _shared(x_bf, shared_gate, shared_up, shared_down):
    SH = shared_gate.shape[0]
    return pl.pallas_call(
        _shared_body,
        grid=(T // _BT4,),
        in_specs=[
            pl.BlockSpec((_BT4, D), lambda i: (i, 0)),
            pl.BlockSpec((SH, D), lambda i: (0, 0)),
            pl.BlockSpec((SH, D), lambda i: (0, 0)),
            pl.BlockSpec((D, SH), lambda i: (0, 0)),
        ],
        out_specs=pl.BlockSpec((_BT4, D), lambda i: (i, 0)),
        out_shape=jax.ShapeDtypeStruct((T, D), jnp.float32),
        interpret=INTERP,
    )(x_bf, shared_gate, shared_up, shared_down)


_BTC = 1024


def _combine_body(rk_ref, w_ref, y_ref, sh_ref, o_ref):
    ir = lax.broadcasted_iota(jnp.int32, (_BTC, CAP), 1)
    li = lax.broadcasted_iota(jnp.int32, (_BTC, E), 1)
    eidx = pl.program_id(1)
    col = jnp.sum(jnp.where(li == eidx, rk_ref[...], 0.0), axis=1,
                  keepdims=True).astype(jnp.int32)
    wcol = jnp.sum(jnp.where(li == eidx, w_ref[...], 0.0), axis=1,
                   keepdims=True)

    @pl.when(pl.program_id(1) == 0)
    def _():
        o_ref[...] = sh_ref[...]

    @pl.when(jnp.any((col >= 0) & (col < CAP)))
    def _():
        a = (col == ir).astype(jnp.bfloat16)
        gath = jnp.dot(a, y_ref[0], preferred_element_type=jnp.float32)
        o_ref[...] = o_ref[...] + wcol * gath


def _combine(y, rankv, wfull, shared_out):
    return pl.pallas_call(
        _combine_body,
        grid=(T // _BTC, E),
        in_specs=[
            pl.BlockSpec((_BTC, E), lambda t, e: (t, 0)),
            pl.BlockSpec((_BTC, E), lambda t, e: (t, 0)),
            pl.BlockSpec((1, CAP, D), lambda t, e: (e, 0, 0)),
            pl.BlockSpec((_BTC, D), lambda t, e: (t, 0)),
        ],
        out_specs=pl.BlockSpec((_BTC, D), lambda t, e: (t, 0)),
        out_shape=jax.ShapeDtypeStruct((T, D), jnp.float32),
        interpret=INTERP,
    )(rankv, wfull, y, shared_out)


def kernel(hidden_states, router_weight, e_bias, W_gate, W_up, W_down,
           shared_gate, shared_up, shared_down):
    B, S, Dm = hidden_states.shape
    x = hidden_states.reshape(T, Dm)
    x_bf = x.astype(jnp.bfloat16)

    wfull, rankv = _router(x, router_weight, e_bias)
    xb = _dispatch(x_bf, rankv)
    y = _mlp(xb.reshape(E, CAP, D), W_gate, W_up, W_down)
    shared_out = _shared(x_bf,
                         shared_gate.astype(jnp.bfloat16),
                         shared_up.astype(jnp.bfloat16),
                         shared_down.astype(jnp.bfloat16))
    out = _combine(y, rankv, wfull, shared_out)
    return out.reshape(B, S, Dm)


# speedup vs baseline: 11.5715x; 1.0017x over previous
import jax
import jax.numpy as jnp
from jax import lax
from jax.experimental import pallas as pl
from jax.experimental.pallas import tpu as pltpu

E = 16
K = 8
D = 2048
I = 1024
N_GROUP = 4
GSZ = E // N_GROUP
ROUTED_SCALING = 2.5
CAP = 640
ECAP = E * CAP
T = 8192

INTERP = False

_NEG = -1e30


_BT = 512


def _router_body(x_ref, rw_ref, eb_ref, w_ref, rk_ref, cnt_ref):
    bt = _BT
    x = x_ref[...]
    logits = lax.dot_general(x, rw_ref[...], (((1,), (1,)), ((), ())),
                             preferred_element_type=jnp.float32)
    scores = jax.nn.sigmoid(logits)
    sfc = scores + eb_ref[...]

    li = lax.broadcasted_iota(jnp.int32, (bt, E), 1)
    lg = li // GSZ

    gs = []
    for g in range(N_GROUP):
        vals = jnp.where(lg == g, sfc, _NEG)
        m1 = jnp.max(vals, axis=1, keepdims=True)
        pos1 = jnp.min(jnp.where(vals == m1, li, 99), axis=1, keepdims=True)
        m2 = jnp.max(jnp.where(li == pos1, _NEG, vals), axis=1, keepdims=True)
        gs.append(m1 + m2)

    best1 = gs[0]
    gi1 = jnp.zeros_like(gs[0], dtype=jnp.int32)
    for g in range(1, N_GROUP):
        b = gs[g] > best1
        best1 = jnp.where(b, gs[g], best1)
        gi1 = jnp.where(b, g, gi1)
    best2 = jnp.full_like(best1, _NEG)
    gi2 = jnp.full_like(gi1, -1)
    for g in range(N_GROUP):
        b = (gi1 != g) & (gs[g] > best2)
        best2 = jnp.where(b, gs[g], best2)
        gi2 = jnp.where(b, g, gi2)
    gmask = (lg == gi1) | (lg == gi2)

    masked = jnp.where(gmask, sfc, 0.0)

    cur = masked
    sel = jnp.zeros((bt, E), dtype=jnp.bool_)
    for _ in range(K):
        m = jnp.max(cur, axis=1, keepdims=True)
        pos = jnp.min(jnp.where(cur == m, li, 99), axis=1, keepdims=True)
        hit = li == pos
        sel = sel | hit
        cur = jnp.where(hit, _NEG, cur)

    selr = sel.astype(jnp.float32)
    wsum = jnp.sum(jnp.where(sel, scores, 0.0), axis=1, keepdims=True)
    inv = ROUTED_SCALING / (wsum + 1e-20)

    @pl.when(pl.program_id(0) == 0)
    def _():
        cnt_ref[...] = jnp.zeros_like(cnt_ref)

    r0 = lax.broadcasted_iota(jnp.int32, (bt, bt), 0)
    r1 = lax.broadcasted_iota(jnp.int32, (bt, bt), 1)
    tri = (r0 > r1).astype(jnp.bfloat16)
    prefix = jnp.dot(tri, selr.astype(jnp.bfloat16),
                     preferred_element_type=jnp.float32)
    rank_f = prefix + cnt_ref[...]
    cnt_ref[...] = cnt_ref[...] + jnp.sum(selr, axis=0, keepdims=True)

    keep = sel & (rank_f < CAP)
    w_ref[...] = jnp.where(keep, scores * inv, 0.0)
    rk_ref[...] = jnp.where(sel, rank_f, -1.0)


def _router(x, router_weight, e_bias):
    return pl.pallas_call(
        _router_body,
        grid=(T // _BT,),
        in_specs=[
            pl.BlockSpec((_BT, D), lambda i: (i, 0)),
            pl.BlockSpec((E, D), lambda i: (0, 0)),
            pl.BlockSpec((1, E), lambda i: (0, 0)),
        ],
        out_specs=[
            pl.BlockSpec((_BT, E), lambda i: (i, 0)),
            pl.BlockSpec((_BT, E), lambda i: (i, 0)),
        ],
        out_shape=[
            jax.ShapeDtypeStruct((T, E), jnp.float32),
            jax.ShapeDtypeStruct((T, E), jnp.float32),
        ],
        scratch_shapes=[pltpu.VMEM((1, E), jnp.float32)],
        interpret=INTERP,
    )(x, router_weight, e_bias.reshape(1, E))


_EG = 4
_NG = E // _EG
_BTD = 512
_NTB = T // _BTD


def _dispatch_body(rk_ref, x_ref, xb_ref):
    x = x_ref[...]
    ir = lax.broadcasted_iota(jnp.int32, (_BTD, CAP), 1)
    li = lax.broadcasted_iota(jnp.int32, (_BTD, E), 1)
    rk = rk_ref[...]
    for eg in range(_EG):
        eidx = pl.program_id(0) * _EG + eg
        col = jnp.sum(jnp.where(li == eidx, rk, 0.0), axis=1,
                      keepdims=True).astype(jnp.int32)
        sl = pl.ds(eg * CAP, CAP)

        @pl.when(pl.program_id(1) == 0)
        def _():
            xb_ref[sl, :] = jnp.zeros((CAP, D), jnp.bfloat16)

        @pl.when(jnp.any((col >= 0) & (col < CAP)))
        def _(col=col):
            m = (col == ir).astype(jnp.bfloat16)
            part = lax.dot_general(m, x, (((0,), (0,)), ((), ())),
                                   preferred_element_type=jnp.float32)
            xb_ref[sl, :] = xb_ref[sl, :] + part.astype(jnp.bfloat16)


def _dispatch(x_bf, rankv):
    return pl.pallas_call(
        _dispatch_body,
        grid=(_NG, _NTB),
        in_specs=[
            pl.BlockSpec((_BTD, E), lambda g, t: (t, 0)),
            pl.BlockSpec((_BTD, D), lambda g, t: (t, 0)),
        ],
        out_specs=pl.BlockSpec((_EG * CAP, D), lambda g, t: (g, 0)),
        out_shape=jax.ShapeDtypeStruct((ECAP, D), jnp.bfloat16),
        interpret=INTERP,
    )(rankv, x_bf)


_NI = 2
_BI = I // _NI


def _mlp_body(xb_ref, wg_ref, wu_ref, wd_ref, y_ref):
    xb = xb_ref[0]
    wg = wg_ref[0].astype(jnp.bfloat16)
    wu = wu_ref[0].astype(jnp.bfloat16)
    wd = wd_ref[0].astype(jnp.bfloat16)
    g = jnp.dot(xb, wg, preferred_element_type=jnp.float32)
    u = jnp.dot(xb, wu, preferred_element_type=jnp.float32)
    h = (g * jax.nn.sigmoid(g) * u).astype(jnp.bfloat16)
    part = jnp.dot(h, wd, preferred_element_type=jnp.float32)

    @pl.when(pl.program_id(1) == 0)
    def _():
        y_ref[0] = part.astype(jnp.bfloat16)

    @pl.when(pl.program_id(1) != 0)
    def _():
        y_ref[0] = (y_ref[0].astype(jnp.float32) + part).astype(jnp.bfloat16)


def _mlp(xb, W_gate, W_up, W_down):
    return pl.pallas_call(
        _mlp_body,
        grid=(E, _NI),
        in_specs=[
            pl.BlockSpec((1, CAP, D), lambda e, s: (e, 0, 0)),
            pl.BlockSpec((1, D, _BI), lambda e, s: (e, 0, s)),
            pl.BlockSpec((1, D, _BI), lambda e, s: (e, 0, s)),
            pl.BlockSpec((1, _BI, D), lambda e, s: (e, s, 0)),
        ],
        out_specs=pl.BlockSpec((1, CAP, D), lambda e, s: (e, 0, 0)),
        out_shape=jax.ShapeDtypeStruct((E, CAP, D), jnp.bfloat16),
        interpret=INTERP,
    )(xb, W_gate, W_up, W_down)


_BT4 = 512


def _shared_body(x_ref, sg_ref, su_ref, sd_ref, o_ref):
    x = x_ref[...]
    g = lax.dot_general(x, sg_ref[...], (((1,), (1,)), ((), ())),
                        preferred_element_type=jnp.float32)
    u = lax.dot_general(x, su_ref[...], (((1,), (1,)), ((), ())),
                        preferred_element_type=jnp.float32)
    h = (g * jax.nn.sigmoid(g) * u).astype(jnp.bfloat16)
    o_ref[...] = lax.dot_general(h, sd_ref[...], (((1,), (1,)), ((), ())),
                                 preferred_element_type=jnp.float32)


def _shared(x_bf, shared_gate, shared_up, shared_down):
    SH = shared_gate.shape[0]
    return pl.pallas_call(
        _shared_body,
        grid=(T // _BT4,),
        in_specs=[
            pl.BlockSpec((_BT4, D), lambda i: (i, 0)),
            pl.BlockSpec((SH, D), lambda i: (0, 0)),
            pl.BlockSpec((SH, D), lambda i: (0, 0)),
            pl.BlockSpec((D, SH), lambda i: (0, 0)),
        ],
        out_specs=pl.BlockSpec((_BT4, D), lambda i: (i, 0)),
        out_shape=jax.ShapeDtypeStruct((T, D), jnp.float32),
        interpret=INTERP,
    )(x_bf, shared_gate, shared_up, shared_down)


_BTC = 1024


def _combine_body(rk_ref, w_ref, y_ref, sh_ref, o_ref):
    ir = lax.broadcasted_iota(jnp.int32, (_BTC, CAP), 1)
    li = lax.broadcasted_iota(jnp.int32, (_BTC, E), 1)
    eidx = pl.program_id(1)
    col = jnp.sum(jnp.where(li == eidx, rk_ref[...], 0.0), axis=1,
                  keepdims=True).astype(jnp.int32)
    wcol = jnp.sum(jnp.where(li == eidx, w_ref[...], 0.0), axis=1,
                   keepdims=True)

    @pl.when(pl.program_id(1) == 0)
    def _():
        o_ref[...] = sh_ref[...]

    @pl.when(jnp.any((col >= 0) & (col < CAP)))
    def _():
        a = (col == ir).astype(jnp.bfloat16)
        gath = jnp.dot(a, y_ref[0], preferred_element_type=jnp.float32)
        o_ref[...] = o_ref[...] + wcol * gath


def _combine(y, rankv, wfull, shared_out):
    return pl.pallas_call(
        _combine_body,
        grid=(T // _BTC, E),
        in_specs=[
            pl.BlockSpec((_BTC, E), lambda t, e: (t, 0)),
            pl.BlockSpec((_BTC, E), lambda t, e: (t, 0)),
            pl.BlockSpec((1, CAP, D), lambda t, e: (e, 0, 0)),
            pl.BlockSpec((_BTC, D), lambda t, e: (t, 0)),
        ],
        out_specs=pl.BlockSpec((_BTC, D), lambda t, e: (t, 0)),
        out_shape=jax.ShapeDtypeStruct((T, D), jnp.float32),
        interpret=INTERP,
    )(rankv, wfull, y, shared_out)


def kernel(hidden_states, router_weight, e_bias, W_gate, W_up, W_down,
           shared_gate, shared_up, shared_down):
    B, S, Dm = hidden_states.shape
    x = hidden_states.reshape(T, Dm)
    x_bf = x.astype(jnp.bfloat16)

    wfull, rankv = _router(x, router_weight, e_bias)
    xb = _dispatch(x_bf, rankv)
    y = _mlp(xb.reshape(E, CAP, D), W_gate, W_up, W_down)
    shared_out = _shared(x_bf,
                         shared_gate.astype(jnp.bfloat16),
                         shared_up.astype(jnp.bfloat16),
                         shared_down.astype(jnp.bfloat16))
    out = _combine(y, rankv, wfull, shared_out)
    return out.reshape(B, S, Dm)
